# Initial kernel scaffold; baseline (speedup 1.0000x reference)
import jax
import jax.numpy as jnp
from jax.experimental import pallas as pl


def kernel(pred, gt, mask):
    raise NotImplementedError("write your pallas kernel here")


# speedup vs baseline: 62.1582x; 62.1582x over previous
import jax
import jax.numpy as jnp
from jax import lax
from jax.experimental import pallas as pl
from jax.experimental.pallas import tpu as pltpu

_R, _C = 2048, 1024
_BR = 256
_G = _R // _BR
_EPS = 1e-06
_NEG_RATIO = 3.0
_HI_BITS = 0x43480000
_BISECT_ITERS = 31


def _sums_body(p_ref, g_ref, m_ref, o_ref):
    i = pl.program_id(0)
    p = p_ref[...]
    g = g_ref[...]
    m = m_ref[...]
    l = -jnp.log(jnp.where(g > 0.5, p, 1.0 - p))
    pos = g * m
    s_pos = jnp.sum(pos)
    s_m = jnp.sum(m)
    pll = jnp.sum(pos * l)
    mll = jnp.sum(m * l)

    @pl.when(i == 0)
    def _init():
        o_ref[0, 0] = s_pos
        o_ref[0, 1] = s_m
        o_ref[0, 2] = pll
        o_ref[0, 3] = mll - pll

    @pl.when(i != 0)
    def _acc():
        o_ref[0, 0] += s_pos
        o_ref[0, 1] += s_m
        o_ref[0, 2] += pll
        o_ref[0, 3] += mll - pll


def _global_sums(p2, g2, m2):
    out = pl.pallas_call(
        _sums_body,
        grid=(_G,),
        in_specs=[pl.BlockSpec((_BR, _C), lambda i: (i, 0))] * 3,
        out_specs=pl.BlockSpec(memory_space=pltpu.SMEM),
        out_shape=jax.ShapeDtypeStruct((1, 4), jnp.float32),
    )(p2, g2, m2)
    return out[0, 0], out[0, 1], out[0, 2], out[0, 3]


def _topk_body(k_ref, p_ref, g_ref, m_ref, o_ref, ib_ref, fb_ref):
    it = pl.program_id(0)
    blk = pl.program_id(1)
    nblk = pl.num_programs(1)
    k = k_ref[0, 0]

    p = p_ref[...]
    g = g_ref[...]
    m = m_ref[...]
    l = -jnp.log(jnp.where(g > 0.5, p, 1.0 - p))
    nl = (m - g * m) * l

    @pl.when(jnp.logical_and(it == 0, blk == 0))
    def _init():
        ib_ref[0] = 0
        ib_ref[1] = _HI_BITS
        fb_ref[0] = 0.0
        fb_ref[1] = 0.0

    @pl.when(jnp.logical_and(it > 0, blk == 0))
    def _update():
        lo = ib_ref[0]
        hi = ib_ref[1]
        mid = lo + (hi - lo) // 2
        took = fb_ref[0] >= k
        ib_ref[0] = jnp.where(took, mid, lo)
        ib_ref[1] = jnp.where(took, hi, mid)
        fb_ref[0] = 0.0

    @pl.when(it < _BISECT_ITERS)
    def _count():
        lo = ib_ref[0]
        hi = ib_ref[1]
        mid = lo + (hi - lo) // 2
        t = lax.bitcast_convert_type(mid, jnp.float32)
        fb_ref[0] += jnp.sum((nl >= t).astype(jnp.float32))

    @pl.when(it == _BISECT_ITERS)
    def _final():
        t = lax.bitcast_convert_type(ib_ref[0], jnp.float32)
        fb_ref[0] += jnp.sum((nl > t).astype(jnp.float32))
        fb_ref[1] += jnp.sum(jnp.where(nl > t, nl, 0.0))

        @pl.when(blk == nblk - 1)
        def _emit():
            o_ref[0, 0] = fb_ref[1] + (k - fb_ref[0]) * t


def _topk_sum(k, p2, g2, m2):
    out = pl.pallas_call(
        _topk_body,
        grid=(_BISECT_ITERS + 1, _G),
        in_specs=[
            pl.BlockSpec(memory_space=pltpu.SMEM),
            pl.BlockSpec((_BR, _C), lambda it, b: (b, 0)),
            pl.BlockSpec((_BR, _C), lambda it, b: (b, 0)),
            pl.BlockSpec((_BR, _C), lambda it, b: (b, 0)),
        ],
        out_specs=pl.BlockSpec(memory_space=pltpu.SMEM),
        out_shape=jax.ShapeDtypeStruct((1, 1), jnp.float32),
        scratch_shapes=[
            pltpu.SMEM((2,), jnp.int32),
            pltpu.SMEM((2,), jnp.float32),
        ],
    )(jnp.reshape(k, (1, 1)), p2, g2, m2)
    return out[0, 0]


def kernel(pred, gt, mask):
    p2 = pred.reshape(_R, _C)
    g2 = gt.reshape(_R, _C)
    m2 = mask.reshape(_R, _C)

    pos_count, mask_count, pos_loss_sum, neg_loss_sum = _global_sums(p2, g2, m2)
    neg_total = mask_count - pos_count
    k = jnp.minimum(neg_total, jnp.floor(pos_count * _NEG_RATIO))

    neg_topk_sum = lax.cond(
        k >= neg_total,
        lambda: neg_loss_sum,
        lambda: lax.cond(
            k <= 0.0,
            lambda: jnp.float32(0.0),
            lambda: _topk_sum(k, p2, g2, m2),
        ),
    )
    return (pos_loss_sum + neg_topk_sum) / (pos_count + k + _EPS)
